# baseline (speedup 1.0000x reference)
import functools

import jax
import jax.numpy as jnp
from jax import lax
from jax.experimental import pallas as pl
from jax.experimental.pallas import tpu as pltpu
from jax.experimental.pallas import tpu_sc as plsc

F = 26
V = 51
D = 64
B = 16384
NC = 2
NS = 16
NW = NC * NS
BPW = B // NW
CHK = 128
NCHK = BPW // CHK
TAB = F * V * D

_mesh = plsc.VectorSubcoreMesh(
    core_axis_name="c", subcore_axis_name="s", num_cores=NC, num_subcores=NS)


@functools.partial(
    pl.kernel,
    out_type=jax.ShapeDtypeStruct((F, D, B), jnp.float32),
    mesh=_mesh,
    scratch_types=[
        pltpu.VMEM((TAB,), jnp.float32),
        pltpu.VMEM((F, BPW), jnp.int32),
        pltpu.VMEM((2, D, CHK), jnp.float32),
        pltpu.SemaphoreType.DMA,
        pltpu.SemaphoreType.DMA,
    ],
    compiler_params=pltpu.CompilerParams(
        use_tc_tiling_on_sc=True, needs_layout_passes=False),
)
def _emb_lookup(tab_hbm, idx_hbm, out_hbm, tab_v, idx_v, blk_v, sem0, sem1):
    wid = lax.axis_index("s") * NC + lax.axis_index("c")
    b0 = wid * BPW

    pltpu.sync_copy(tab_hbm, tab_v)
    pltpu.sync_copy(idx_hbm.at[wid], idx_v)

    sems = (sem0, sem1)

    def stage(f, h, slot):
        base = f * (V * D)
        for i in range(CHK // 16):
            idx16 = idx_v[f, pl.ds(h * CHK + i * 16, 16)]
            flat = idx16 * D + base
            def body(d):
                blk_v[slot, d, pl.ds(i * 16, 16)] = plsc.load_gather(
                    tab_v, [flat + d])
            pl.loop(0, D, unroll=8)(body)
        pltpu.async_copy(
            blk_v.at[slot],
            out_hbm.at[f, :, pl.ds(b0 + h * CHK, CHK)],
            sems[slot])

    def wait(slot):
        pltpu.make_async_copy(
            blk_v.at[slot],
            out_hbm.at[0, :, pl.ds(b0, CHK)],
            sems[slot]).wait()

    def run_f(f):
        for h in range(NCHK):
            slot = h % 2
            if h < 2:
                @pl.when(f > 0)
                def _():
                    wait(slot)
            else:
                wait(slot)
            stage(f, h, slot)
    pl.loop(0, F)(run_f)

    wait(0)
    wait(1)


def kernel(x, tables):
    tab_flat = tables.reshape(TAB)
    xw = x.T.reshape(F, NW, BPW).transpose(1, 0, 2)
    out_t = _emb_lookup(tab_flat, xw)
    return jnp.transpose(out_t, (2, 0, 1))


# speedup vs baseline: 6.5789x; 6.5789x over previous
import functools

import jax
import jax.numpy as jnp
from jax import lax
from jax.experimental import pallas as pl
from jax.experimental.pallas import tpu as pltpu
from jax.experimental.pallas import tpu_sc as plsc

F = 26
V = 51
D = 64
B = 16384
NC = 2
NS = 16
NW = NC * NS
BPW = B // NW
CHK = 128
NCHK = BPW // CHK
VR = D + 1
TAB = F * V * VR

_mesh = plsc.VectorSubcoreMesh(
    core_axis_name="c", subcore_axis_name="s", num_cores=NC, num_subcores=NS)


@functools.partial(
    pl.kernel,
    out_type=jax.ShapeDtypeStruct((F, D, B), jnp.float32),
    mesh=_mesh,
    scratch_types=[
        pltpu.VMEM((TAB,), jnp.float32),
        pltpu.VMEM((F, BPW), jnp.int32),
        pltpu.VMEM((2, D, CHK), jnp.float32),
        pltpu.SemaphoreType.DMA,
        pltpu.SemaphoreType.DMA,
    ],
    compiler_params=pltpu.CompilerParams(
        use_tc_tiling_on_sc=True, needs_layout_passes=False),
)
def _emb_lookup(tab_hbm, idx_hbm, out_hbm, tab_v, idx_v, blk_v, sem0, sem1):
    wid = lax.axis_index("s") * NC + lax.axis_index("c")
    b0 = wid * BPW

    pltpu.sync_copy(tab_hbm, tab_v)
    pltpu.sync_copy(idx_hbm.at[wid], idx_v)

    sems = (sem0, sem1)

    def stage(f, h, slot, basev):
        for i in range(CHK // 16):
            idx16 = idx_v[f, pl.ds(h * CHK + i * 16, 16)]
            flatv = idx16 * VR + basev
            def dbody(d):
                blk_v[slot, d, pl.ds(i * 16, 16)] = plsc.load_gather(
                    tab_v, [flatv + d])
            plsc.parallel_loop(0, D, unroll=8)(dbody)
        pltpu.async_copy(
            blk_v.at[slot],
            out_hbm.at[f, :, pl.ds(b0 + h * CHK, CHK)],
            sems[slot])

    def wait(slot):
        pltpu.make_async_copy(
            blk_v.at[slot],
            out_hbm.at[0, :, pl.ds(b0, CHK)],
            sems[slot]).wait()

    def run_f(f):
        basev = jnp.full((16,), f * (V * VR), dtype=jnp.int32)
        for h in range(NCHK):
            slot = h % 2
            if h < 2:
                @pl.when(f > 0)
                def _():
                    wait(slot)
            else:
                wait(slot)
            stage(f, h, slot, basev)
    pl.loop(0, F)(run_f)

    wait(0)
    wait(1)


def kernel(x, tables):
    tab_flat = jnp.pad(tables, ((0, 0), (0, 0), (0, VR - D))).reshape(TAB)
    xw = x.T.reshape(F, NW, BPW).transpose(1, 0, 2)
    out_t = _emb_lookup(tab_flat, xw)
    return jnp.transpose(out_t, (2, 0, 1))


# speedup vs baseline: 9.2323x; 1.4033x over previous
import functools

import jax
import jax.numpy as jnp
from jax import lax
from jax.experimental import pallas as pl
from jax.experimental.pallas import tpu as pltpu
from jax.experimental.pallas import tpu_sc as plsc

F = 26
V = 51
D = 64
B = 16384
NC = 2
NS = 16
NW = NC * NS
BPW = B // NW
CHK = 128
NCHK = BPW // CHK
VR = D + 1
TAB = F * V * VR

_mesh = plsc.VectorSubcoreMesh(
    core_axis_name="c", subcore_axis_name="s", num_cores=NC, num_subcores=NS)


@functools.partial(
    pl.kernel,
    out_type=jax.ShapeDtypeStruct((F, D, B), jnp.float32),
    mesh=_mesh,
    scratch_types=[
        pltpu.VMEM((TAB,), jnp.float32),
        pltpu.VMEM((F, BPW), jnp.int32),
        pltpu.VMEM((2, D, CHK), jnp.float32),
        pltpu.SemaphoreType.DMA,
        pltpu.SemaphoreType.DMA,
    ],
    compiler_params=pltpu.CompilerParams(
        use_tc_tiling_on_sc=True, needs_layout_passes=False),
)
def _emb_lookup(tab_hbm, idx_hbm, out_hbm, tab_v, idx_v, blk_v, sem0, sem1):
    wid = lax.axis_index("s") * NC + lax.axis_index("c")
    b0 = wid * BPW

    pltpu.sync_copy(tab_hbm, tab_v)
    pltpu.sync_copy(idx_hbm.at[:, pl.ds(b0, BPW)], idx_v)

    sems = (sem0, sem1)

    def stage(f, h, slot, basev):
        for i0 in range(0, CHK // 16, 4):
            flats = []
            for g in range(4):
                idx16 = idx_v[f, pl.ds(h * CHK + (i0 + g) * 16, 16)]
                flats.append(idx16 * VR + basev)
            def dbody(d):
                for g in range(4):
                    blk_v[slot, d, pl.ds((i0 + g) * 16, 16)] = (
                        plsc.load_gather(tab_v, [flats[g] + d]))
            plsc.parallel_loop(0, D, unroll=4)(dbody)
        pltpu.async_copy(
            blk_v.at[slot],
            out_hbm.at[f, :, pl.ds(b0 + h * CHK, CHK)],
            sems[slot])

    def wait(slot):
        pltpu.make_async_copy(
            blk_v.at[slot],
            out_hbm.at[0, :, pl.ds(b0, CHK)],
            sems[slot]).wait()

    def run_f(f):
        basev = jnp.full((16,), f * (V * VR), dtype=jnp.int32)
        for h in range(NCHK):
            slot = h % 2
            if h < 2:
                @pl.when(f > 0)
                def _():
                    wait(slot)
            else:
                wait(slot)
            stage(f, h, slot, basev)
    pl.loop(0, F)(run_f)

    wait(0)
    wait(1)


def kernel(x, tables):
    tab_flat = jnp.pad(tables, ((0, 0), (0, 0), (0, VR - D))).reshape(TAB)
    out_t = _emb_lookup(tab_flat, x.T)
    return jnp.transpose(out_t, (2, 0, 1))


# speedup vs baseline: 9.4628x; 1.0250x over previous
import functools

import jax
import jax.numpy as jnp
from jax import lax
from jax.experimental import pallas as pl
from jax.experimental.pallas import tpu as pltpu
from jax.experimental.pallas import tpu_sc as plsc

F = 26
V = 51
D = 64
B = 16384
NC = 2
NS = 16
NW = NC * NS
BPW = B // NW
CHK = 128
NCHK = BPW // CHK
VR = D + 1
TAB = F * V * VR
FSPLIT = 16

_mesh = plsc.VectorSubcoreMesh(
    core_axis_name="c", subcore_axis_name="s", num_cores=NC, num_subcores=NS)


@functools.partial(
    pl.kernel,
    out_type=jax.ShapeDtypeStruct((F, D, B), jnp.float32),
    mesh=_mesh,
    scratch_types=[
        pltpu.VMEM((TAB,), jnp.float32),
        pltpu.VMEM((F, BPW), jnp.int32),
        pltpu.VMEM((2, D, CHK), jnp.float32),
        pltpu.SemaphoreType.DMA,
        pltpu.SemaphoreType.DMA,
        pltpu.SemaphoreType.DMA,
    ],
    compiler_params=pltpu.CompilerParams(
        use_tc_tiling_on_sc=True, needs_layout_passes=False),
)
def _emb_lookup(tab_hbm, idx_hbm, out_hbm, tab_v, idx_v, blk_v,
                sem0, sem1, semt):
    wid = lax.axis_index("s") * NC + lax.axis_index("c")
    b0 = wid * BPW

    h1 = FSPLIT * V * VR
    pltpu.async_copy(tab_hbm.at[pl.ds(0, h1)], tab_v.at[pl.ds(0, h1)], semt)
    pltpu.async_copy(
        tab_hbm.at[pl.ds(h1, TAB - h1)], tab_v.at[pl.ds(h1, TAB - h1)], semt)
    pltpu.sync_copy(idx_hbm.at[:, pl.ds(b0, BPW)], idx_v)
    pltpu.make_async_copy(
        tab_hbm.at[pl.ds(0, h1)], tab_v.at[pl.ds(0, h1)], semt).wait()

    sems = (sem0, sem1)

    def stage(f, h, slot, basev):
        for i0 in range(0, CHK // 16, 4):
            flats = []
            for g in range(4):
                idx16 = idx_v[f, pl.ds(h * CHK + (i0 + g) * 16, 16)]
                flats.append(idx16 * VR + basev)
            def dbody(d):
                for g in range(4):
                    blk_v[slot, d, pl.ds((i0 + g) * 16, 16)] = (
                        plsc.load_gather(tab_v, [flats[g] + d]))
            plsc.parallel_loop(0, D, unroll=4)(dbody)
        pltpu.async_copy(
            blk_v.at[slot],
            out_hbm.at[f, :, pl.ds(b0 + h * CHK, CHK)],
            sems[slot])

    def wait(slot):
        pltpu.make_async_copy(
            blk_v.at[slot],
            out_hbm.at[0, :, pl.ds(b0, CHK)],
            sems[slot]).wait()

    def run_f(f):
        @pl.when(f == FSPLIT)
        def _():
            pltpu.make_async_copy(
                tab_hbm.at[pl.ds(h1, TAB - h1)],
                tab_v.at[pl.ds(h1, TAB - h1)], semt).wait()
        basev = jnp.full((16,), f * (V * VR), dtype=jnp.int32)
        for h in range(NCHK):
            slot = h % 2
            if h < 2:
                @pl.when(f > 0)
                def _():
                    wait(slot)
            else:
                wait(slot)
            stage(f, h, slot, basev)
    pl.loop(0, F)(run_f)

    wait(0)
    wait(1)


def kernel(x, tables):
    tab_flat = jnp.pad(tables, ((0, 0), (0, 0), (0, VR - D))).reshape(TAB)
    out_t = _emb_lookup(tab_flat, x.T)
    return jnp.transpose(out_t, (2, 0, 1))
